# Initial kernel scaffold; baseline (speedup 1.0000x reference)
import jax
import jax.numpy as jnp
from jax.experimental import pallas as pl


def kernel(input, cb0, cb1, cb2, cb3):
    raise NotImplementedError("write your pallas kernel here")


# speedup vs baseline: 3.0419x; 3.0419x over previous
import functools

import numpy as np
import jax
import jax.numpy as jnp
from jax import lax
from jax.experimental import pallas as pl
from jax.experimental.pallas import tpu as pltpu
from jax.experimental.pallas import tpu_sc as plsc

K = 1024
CH = 256
B = 4
H = W = 32
R0 = B * H * W
R1 = B * (H // 2) * (W // 2)
R2 = B * (H // 4) * (W // 4)
BLK = 512
NCB0 = (R0 + R1 + R2 + BLK - 1) // BLK
NROWS = NCB0 * BLK + R0
GRID = NROWS // BLK

NC, NS, LANES = 2, 16, 16
NW = NC * NS
BPW = R0 // NW


def _match_body(rows_ref, cbt_ref, cb2_ref, x2_ref, p_ref, idx_ref):
    j = pl.program_id(0)
    rows = rows_ref[...]
    prod = lax.dot_general(rows, cbt_ref[0], (((1,), (0,)), ((), ())),
                           preferred_element_type=jnp.float32)
    dist = x2_ref[...] - 2.0 * prod + cb2_ref[0]
    dmin = jnp.min(dist, axis=1, keepdims=True)
    ii = lax.broadcasted_iota(jnp.int32, (BLK, K), 1)
    idx_ref[...] = jnp.min(jnp.where(dist == dmin, ii, K), axis=1, keepdims=True)
    p_ref[...] = jnp.zeros((BLK, 1), jnp.float32)

    @pl.when(j < NCB0)
    def _():
        denom = jnp.sum(jnp.exp(dmin - dist), axis=1, keepdims=True)
        p_ref[...] = 1.0 / denom


def _tc_match(rows, cbts, cb2s, x2):
    return pl.pallas_call(
        _match_body,
        grid=(GRID,),
        in_specs=[
            pl.BlockSpec((BLK, CH), lambda j: (j, 0)),
            pl.BlockSpec((1, CH, K), lambda j: (j // NCB0, 0, 0)),
            pl.BlockSpec((1, 1, K), lambda j: (j // NCB0, 0, 0)),
            pl.BlockSpec((BLK, 1), lambda j: (j, 0)),
        ],
        out_specs=[
            pl.BlockSpec((BLK, 1), lambda j: (j, 0)),
            pl.BlockSpec((BLK, 1), lambda j: (j, 0)),
        ],
        out_shape=[
            jax.ShapeDtypeStruct((NROWS, 1), jnp.float32),
            jax.ShapeDtypeStruct((NROWS, 1), jnp.int32),
        ],
    )(rows, cbts, cb2s, x2)


def _upsample_maps():
    b, h, w = np.meshgrid(np.arange(B), np.arange(H), np.arange(W), indexing='ij')
    m1 = b * (H // 2) * (W // 2) + (h // 2) * (W // 2) + (w // 2)
    m2 = b * (H // 4) * (W // 4) + (h // 4) * (W // 4) + (w // 4)
    return (jnp.asarray(m1.reshape(-1), jnp.int32),
            jnp.asarray(m2.reshape(-1), jnp.int32))


def _sc_body(p0_h, i0_h, p1_h, i1_h, p2_h, i2_h, z2_h, m1_h, m2_h,
             cb0h_h, cb1h_h, zout_h, q_h,
             p0_v, i0_v, z2_v, m1_v, m2_v, p1g_v, i1g_v, p2g_v, i2g_v,
             zsel_v, rows0_v, rows1_v, sem):
    wid = lax.axis_index("s") * NC + lax.axis_index("c")
    base = wid * BPW
    pltpu.sync_copy(p0_h.at[pl.ds(base, BPW)], p0_v)
    pltpu.sync_copy(i0_h.at[pl.ds(base, BPW)], i0_v)
    pltpu.sync_copy(z2_h.at[pl.ds(base, BPW)], z2_v)
    pltpu.sync_copy(m1_h.at[pl.ds(base, BPW)], m1_v)
    pltpu.sync_copy(m2_h.at[pl.ds(base, BPW)], m2_v)
    pltpu.async_copy(p1_h.at[m1_v], p1g_v, sem).wait()
    pltpu.async_copy(i1_h.at[m1_v], i1g_v, sem).wait()
    pltpu.async_copy(p2_h.at[m2_v], p2g_v, sem).wait()
    pltpu.async_copy(i2_h.at[m2_v], i2g_v, sem).wait()
    for jc in range(BPW // LANES):
        sl = pl.ds(jc * LANES, LANES)
        p0c = p0_v[sl]
        i0c = i0_v[sl]
        p1c = p1g_v[sl]
        i1c = i1g_v[sl]
        p2c = p2g_v[sl]
        i2c = i2g_v[sl]
        c1 = p1c > p0c
        best = jnp.where(c1, p1c, p0c)
        bidx = jnp.where(c1, i1c, i0c)
        c2 = p2c > best
        zsel_v[sl] = jnp.where(c2, i2c, bidx)
    pltpu.async_copy(cb0h_h.at[zsel_v], rows0_v, sem).wait()
    pltpu.async_copy(cb1h_h.at[z2_v], rows1_v, sem).wait()

    def _add_row(r, carry):
        for c in range(CH // LANES):
            s2 = pl.ds(c * LANES, LANES)
            rows0_v[r, s2] = rows0_v[r, s2] + rows1_v[r, s2]
        return carry

    lax.fori_loop(0, BPW, _add_row, 0)
    pltpu.sync_copy(zsel_v, zout_h.at[pl.ds(base, BPW)])
    pltpu.sync_copy(rows0_v, q_h.at[pl.ds(base, BPW)])


def _sc_combine(p0, i0, p1, i1, p2, i2, z2, m1, m2, cb0h, cb1h):
    mesh = plsc.VectorSubcoreMesh(core_axis_name="c", subcore_axis_name="s",
                                  num_cores=NC, num_subcores=NS)
    fn = pl.kernel(
        _sc_body,
        out_type=(jax.ShapeDtypeStruct((R0,), jnp.int32),
                  jax.ShapeDtypeStruct((R0, CH), jnp.float32)),
        mesh=mesh,
        scratch_types=[
            pltpu.VMEM((BPW,), jnp.float32),
            pltpu.VMEM((BPW,), jnp.int32),
            pltpu.VMEM((BPW,), jnp.int32),
            pltpu.VMEM((BPW,), jnp.int32),
            pltpu.VMEM((BPW,), jnp.int32),
            pltpu.VMEM((BPW,), jnp.float32),
            pltpu.VMEM((BPW,), jnp.int32),
            pltpu.VMEM((BPW,), jnp.float32),
            pltpu.VMEM((BPW,), jnp.int32),
            pltpu.VMEM((BPW,), jnp.int32),
            pltpu.VMEM((BPW, CH), jnp.float32),
            pltpu.VMEM((BPW, CH), jnp.float32),
            pltpu.SemaphoreType.DMA,
        ],
    )
    return fn(p0, i0, p1, i1, p2, i2, z2, m1, m2, cb0h, cb1h)


def kernel(input, cb0, cb1, cb2, cb3):
    b, c, h, w = input.shape
    e0 = jnp.transpose(input, (0, 2, 3, 1)).reshape(R0, CH)
    r1 = jax.image.resize(input, (b, c, h // 2, w // 2), method='bilinear')
    r2 = jax.image.resize(input, (b, c, h // 4, w // 4), method='bilinear')
    e1 = jnp.transpose(r1, (0, 2, 3, 1)).reshape(R1, CH)
    e2 = jnp.transpose(r2, (0, 2, 3, 1)).reshape(R2, CH)

    pad = jnp.zeros((NCB0 * BLK - (R0 + R1 + R2), CH), jnp.float32)
    rows = jnp.concatenate([e0, e1, e2, pad, e0], axis=0)
    x2 = jnp.sum(rows * rows, axis=-1, keepdims=True)
    cbts = jnp.stack([cb0.T, cb1.T], axis=0)
    cb2s = jnp.stack([jnp.sum(cb0 * cb0, axis=-1),
                      jnp.sum(cb1 * cb1, axis=-1)], axis=0).reshape(2, 1, K)

    p, idx = _tc_match(rows, cbts, cb2s, x2)
    p = p.reshape(NROWS)
    idx = idx.reshape(NROWS)
    p0, i0 = p[:R0], idx[:R0]
    p1, i1 = p[R0:R0 + R1], idx[R0:R0 + R1]
    p2, i2 = p[R0 + R1:R0 + R1 + R2], idx[R0 + R1:R0 + R1 + R2]
    z2 = idx[NCB0 * BLK:]

    m1, m2 = _upsample_maps()
    zsel, q = _sc_combine(p0, i0, p1, i1, p2, i2, z2, m1, m2,
                          cb0 * 0.5, cb1 * 0.5)

    zidx0 = jnp.stack([zsel.reshape(b, h, w), z2.reshape(b, h, w)], axis=1)
    quant0 = jnp.transpose(q.reshape(b, h, w, CH), (0, 3, 1, 2))
    return input, zidx0, quant0


# speedup vs baseline: 3.6834x; 1.2109x over previous
import functools

import numpy as np
import jax
import jax.numpy as jnp
from jax import lax
from jax.experimental import pallas as pl
from jax.experimental.pallas import tpu as pltpu
from jax.experimental.pallas import tpu_sc as plsc

K = 1024
CH = 256
B = 4
H = W = 32
N0 = H * W
N1 = N0 // 4
N2 = N0 // 16
R0 = B * N0
R1 = B * N1
R2 = B * N2

NC, NS, LANES = 2, 16, 16
NW = NC * NS
BPW = R0 // NW


def _match_body(x0_ref, x1_ref, x2s_ref, cb0_ref, cb1_ref, c20_ref, c21_ref,
                n0_ref, n1_ref, n2_ref,
                p0_ref, i0_ref, p1_ref, i1_ref, p2_ref, i2_ref, zb_ref):
    def scale(xs, x2row, p_ref, i_ref, cb_ref, c2_ref, need_p):
        n = xs.shape[1]
        prod = lax.dot_general(cb_ref[...], xs, (((1,), (0,)), ((), ())),
                               preferred_element_type=jnp.float32)
        dist = x2row - 2.0 * prod + c2_ref[...]
        dmin = jnp.min(dist, axis=0, keepdims=True)
        ii = lax.broadcasted_iota(jnp.int32, (K, n), 0)
        i_ref[0] = jnp.min(jnp.where(dist == dmin, ii, K), axis=0, keepdims=True)
        if need_p:
            denom = jnp.sum(jnp.exp(dmin - dist), axis=0, keepdims=True)
            p_ref[0] = 1.0 / denom

    scale(x0_ref[0], n0_ref[0], p0_ref, i0_ref, cb0_ref, c20_ref, True)
    scale(x1_ref[0], n1_ref[0], p1_ref, i1_ref, cb0_ref, c20_ref, True)
    scale(x2s_ref[0], n2_ref[0], p2_ref, i2_ref, cb0_ref, c20_ref, True)
    scale(x0_ref[0], n0_ref[0], None, zb_ref, cb1_ref, c21_ref, False)


def _tc_match(x0, x1, x2s, cb0, cb1, c20, c21, norm0, norm1, norm2):
    full = lambda shape: pl.BlockSpec(shape, lambda b: (0,) * len(shape))
    per_b = lambda shape: pl.BlockSpec((1,) + shape, lambda b: (b, 0, 0))
    return pl.pallas_call(
        _match_body,
        grid=(B,),
        in_specs=[
            per_b((CH, N0)), per_b((CH, N1)), per_b((CH, N2)),
            full((K, CH)), full((K, CH)), full((K, 1)), full((K, 1)),
            per_b((1, N0)), per_b((1, N1)), per_b((1, N2)),
        ],
        out_specs=[
            per_b((1, N0)), per_b((1, N0)),
            per_b((1, N1)), per_b((1, N1)),
            per_b((1, N2)), per_b((1, N2)),
            per_b((1, N0)),
        ],
        out_shape=[
            jax.ShapeDtypeStruct((B, 1, N0), jnp.float32),
            jax.ShapeDtypeStruct((B, 1, N0), jnp.int32),
            jax.ShapeDtypeStruct((B, 1, N1), jnp.float32),
            jax.ShapeDtypeStruct((B, 1, N1), jnp.int32),
            jax.ShapeDtypeStruct((B, 1, N2), jnp.float32),
            jax.ShapeDtypeStruct((B, 1, N2), jnp.int32),
            jax.ShapeDtypeStruct((B, 1, N0), jnp.int32),
        ],
    )(x0, x1, x2s, cb0, cb1, c20, c21, norm0, norm1, norm2)


def _upsample_maps():
    b, h, w = np.meshgrid(np.arange(B), np.arange(H), np.arange(W), indexing='ij')
    m1 = b * N1 + (h // 2) * (W // 2) + (w // 2)
    m2 = b * N2 + (h // 4) * (W // 4) + (w // 4)
    return (jnp.asarray(m1.reshape(-1), jnp.int32),
            jnp.asarray(m2.reshape(-1), jnp.int32))


def _sc_body(p0_h, i0_h, p1_h, i1_h, p2_h, i2_h, z2_h, m1_h, m2_h,
             cb0h_h, cb1h_h, zout_h, q_h,
             p0_v, i0_v, z2_v, m1_v, m2_v, p1g_v, i1g_v, p2g_v, i2g_v,
             zsel_v, rows0_v, rows1_v, sem_s, sem_a, sem_b):
    wid = lax.axis_index("s") * NC + lax.axis_index("c")
    base = wid * BPW
    pltpu.sync_copy(z2_h.at[pl.ds(base, BPW)], z2_v)
    cp_b = pltpu.async_copy(cb1h_h.at[z2_v], rows1_v, sem_b)
    pltpu.sync_copy(m1_h.at[pl.ds(base, BPW)], m1_v)
    pltpu.sync_copy(m2_h.at[pl.ds(base, BPW)], m2_v)
    g1 = pltpu.async_copy(p1_h.at[m1_v], p1g_v, sem_s)
    g2 = pltpu.async_copy(i1_h.at[m1_v], i1g_v, sem_s)
    g3 = pltpu.async_copy(p2_h.at[m2_v], p2g_v, sem_s)
    g4 = pltpu.async_copy(i2_h.at[m2_v], i2g_v, sem_s)
    pltpu.sync_copy(p0_h.at[pl.ds(base, BPW)], p0_v)
    pltpu.sync_copy(i0_h.at[pl.ds(base, BPW)], i0_v)
    g1.wait()
    g2.wait()
    g3.wait()
    g4.wait()
    for jc in range(BPW // LANES):
        sl = pl.ds(jc * LANES, LANES)
        p0c = p0_v[sl]
        c1 = p1g_v[sl] > p0c
        best = jnp.where(c1, p1g_v[sl], p0c)
        bidx = jnp.where(c1, i1g_v[sl], i0_v[sl])
        c2 = p2g_v[sl] > best
        zsel_v[sl] = jnp.where(c2, i2g_v[sl], bidx)
    cp_a = pltpu.async_copy(cb0h_h.at[zsel_v], rows0_v, sem_a)
    pltpu.sync_copy(zsel_v, zout_h.at[pl.ds(base, BPW)])
    cp_a.wait()
    cp_b.wait()

    def _add_row(r, carry):
        for c in range(CH // LANES):
            s2 = pl.ds(c * LANES, LANES)
            rows0_v[r, s2] = rows0_v[r, s2] + rows1_v[r, s2]
        return carry

    lax.fori_loop(0, BPW, _add_row, 0)
    pltpu.sync_copy(rows0_v, q_h.at[pl.ds(base, BPW)])


def _sc_combine(p0, i0, p1, i1, p2, i2, z2, m1, m2, cb0h, cb1h):
    mesh = plsc.VectorSubcoreMesh(core_axis_name="c", subcore_axis_name="s",
                                  num_cores=NC, num_subcores=NS)
    fn = pl.kernel(
        _sc_body,
        out_type=(jax.ShapeDtypeStruct((R0,), jnp.int32),
                  jax.ShapeDtypeStruct((R0, CH), jnp.float32)),
        mesh=mesh,
        scratch_types=[
            pltpu.VMEM((BPW,), jnp.float32),
            pltpu.VMEM((BPW,), jnp.int32),
            pltpu.VMEM((BPW,), jnp.int32),
            pltpu.VMEM((BPW,), jnp.int32),
            pltpu.VMEM((BPW,), jnp.int32),
            pltpu.VMEM((BPW,), jnp.float32),
            pltpu.VMEM((BPW,), jnp.int32),
            pltpu.VMEM((BPW,), jnp.float32),
            pltpu.VMEM((BPW,), jnp.int32),
            pltpu.VMEM((BPW,), jnp.int32),
            pltpu.VMEM((BPW, CH), jnp.float32),
            pltpu.VMEM((BPW, CH), jnp.float32),
            pltpu.SemaphoreType.DMA,
            pltpu.SemaphoreType.DMA,
            pltpu.SemaphoreType.DMA,
        ],
    )
    return fn(p0, i0, p1, i1, p2, i2, z2, m1, m2, cb0h, cb1h)


def kernel(input, cb0, cb1, cb2, cb3):
    b, c, h, w = input.shape
    r1 = jax.image.resize(input, (b, c, h // 2, w // 2), method='bilinear')
    r2 = jax.image.resize(input, (b, c, h // 4, w // 4), method='bilinear')
    x0 = input.reshape(B, CH, N0)
    x1 = r1.reshape(B, CH, N1)
    x2s = r2.reshape(B, CH, N2)

    norm0 = jnp.sum(jnp.square(jnp.transpose(input, (0, 2, 3, 1))),
                    axis=-1).reshape(B, 1, N0)
    norm1 = jnp.sum(jnp.square(jnp.transpose(r1, (0, 2, 3, 1))),
                    axis=-1).reshape(B, 1, N1)
    norm2 = jnp.sum(jnp.square(jnp.transpose(r2, (0, 2, 3, 1))),
                    axis=-1).reshape(B, 1, N2)
    c20 = jnp.sum(cb0 * cb0, axis=-1).reshape(K, 1)
    c21 = jnp.sum(cb1 * cb1, axis=-1).reshape(K, 1)

    p0, i0, p1, i1, p2, i2, zb = _tc_match(
        x0, x1, x2s, cb0, cb1, c20, c21, norm0, norm1, norm2)

    m1, m2 = _upsample_maps()
    zsel, q = _sc_combine(
        p0.reshape(R0), i0.reshape(R0), p1.reshape(R1), i1.reshape(R1),
        p2.reshape(R2), i2.reshape(R2), zb.reshape(R0), m1, m2,
        cb0 * 0.5, cb1 * 0.5)

    zidx0 = jnp.stack([zsel.reshape(b, h, w), zb.reshape(b, h, w)], axis=1)
    quant0 = jnp.transpose(q.reshape(b, h, w, CH), (0, 3, 1, 2))
    return input, zidx0, quant0


# speedup vs baseline: 3.6994x; 1.0044x over previous
import functools

import numpy as np
import jax
import jax.numpy as jnp
from jax import lax
from jax.experimental import pallas as pl
from jax.experimental.pallas import tpu as pltpu
from jax.experimental.pallas import tpu_sc as plsc

K = 1024
CH = 256
B = 4
H = W = 32
N0 = H * W
N1 = N0 // 4
N2 = N0 // 16
R0 = B * N0
R1 = B * N1
R2 = B * N2

NC, NS, LANES = 2, 16, 16
NW = NC * NS
BPW = R0 // NW


def _match_body(x0_ref, x1_ref, x2s_ref, cb0_ref, cb1_ref, c20_ref, c21_ref,
                n0_ref, n1_ref, n2_ref,
                p0_ref, i0_ref, p1_ref, i1_ref, p2_ref, i2_ref, zb_ref):
    def scale(xs, x2row, p_ref, i_ref, cb_ref, c2_ref, need_p, flat=True):
        n = xs.shape[1]
        prod = lax.dot_general(cb_ref[...], xs, (((1,), (0,)), ((), ())),
                               preferred_element_type=jnp.float32)
        dist = x2row - 2.0 * prod + c2_ref[...]
        dmin = jnp.min(dist, axis=0, keepdims=True)
        ii = lax.broadcasted_iota(jnp.int32, (K, n), 0)
        idxrow = jnp.min(jnp.where(dist == dmin, ii, K), axis=0, keepdims=True)
        if flat:
            i_ref[...] = jnp.reshape(idxrow, (n,))
        else:
            i_ref[0] = idxrow
        if need_p:
            denom = jnp.sum(jnp.exp(dmin - dist), axis=0, keepdims=True)
            if flat:
                p_ref[...] = jnp.reshape(1.0 / denom, (n,))
            else:
                p_ref[0] = 1.0 / denom

    scale(x0_ref[0], n0_ref[0], p0_ref, i0_ref, cb0_ref, c20_ref, True)
    scale(x1_ref[0], n1_ref[0], p1_ref, i1_ref, cb0_ref, c20_ref, True)
    scale(x2s_ref[0], n2_ref[0], p2_ref, i2_ref, cb0_ref, c20_ref, True,
          flat=False)
    scale(x0_ref[0], n0_ref[0], None, zb_ref, cb1_ref, c21_ref, False)


def _tc_match(x0, x1, x2s, cb0, cb1, c20, c21, norm0, norm1, norm2):
    full = lambda shape: pl.BlockSpec(shape, lambda b: (0,) * len(shape))
    per_b = lambda shape: pl.BlockSpec((1,) + shape, lambda b: (b, 0, 0))
    lin = lambda n: pl.BlockSpec((n,), lambda b: (b,))
    return pl.pallas_call(
        _match_body,
        grid=(B,),
        in_specs=[
            per_b((CH, N0)), per_b((CH, N1)), per_b((CH, N2)),
            full((K, CH)), full((K, CH)), full((K, 1)), full((K, 1)),
            per_b((1, N0)), per_b((1, N1)), per_b((1, N2)),
        ],
        out_specs=[
            lin(N0), lin(N0),
            lin(N1), lin(N1),
            per_b((1, N2)), per_b((1, N2)),
            lin(N0),
        ],
        out_shape=[
            jax.ShapeDtypeStruct((R0,), jnp.float32),
            jax.ShapeDtypeStruct((R0,), jnp.int32),
            jax.ShapeDtypeStruct((R1,), jnp.float32),
            jax.ShapeDtypeStruct((R1,), jnp.int32),
            jax.ShapeDtypeStruct((B, 1, N2), jnp.float32),
            jax.ShapeDtypeStruct((B, 1, N2), jnp.int32),
            jax.ShapeDtypeStruct((R0,), jnp.int32),
        ],
    )(x0, x1, x2s, cb0, cb1, c20, c21, norm0, norm1, norm2)


def _upsample_maps():
    b, h, w = np.meshgrid(np.arange(B), np.arange(H), np.arange(W), indexing='ij')
    m1 = b * N1 + (h // 2) * (W // 2) + (w // 2)
    m2 = b * N2 + (h // 4) * (W // 4) + (w // 4)
    return (jnp.asarray(m1.reshape(-1), jnp.int32),
            jnp.asarray(m2.reshape(-1), jnp.int32))


def _sc_body(p0_h, i0_h, p1_h, i1_h, p2_h, i2_h, z2_h, m1_h, m2_h,
             cb0h_h, cb1h_h, zout_h, q_h,
             p0_v, i0_v, z2_v, m1_v, m2_v, p1g_v, i1g_v, p2g_v, i2g_v,
             zsel_v, rows0_v, rows1_v, sem_s, sem_a, sem_b):
    wid = lax.axis_index("s") * NC + lax.axis_index("c")
    base = wid * BPW
    pltpu.sync_copy(z2_h.at[pl.ds(base, BPW)], z2_v)
    cp_b = pltpu.async_copy(cb1h_h.at[z2_v], rows1_v, sem_b)
    pltpu.sync_copy(m1_h.at[pl.ds(base, BPW)], m1_v)
    pltpu.sync_copy(m2_h.at[pl.ds(base, BPW)], m2_v)
    g1 = pltpu.async_copy(p1_h.at[m1_v], p1g_v, sem_s)
    g2 = pltpu.async_copy(i1_h.at[m1_v], i1g_v, sem_s)
    g3 = pltpu.async_copy(p2_h.at[m2_v], p2g_v, sem_s)
    g4 = pltpu.async_copy(i2_h.at[m2_v], i2g_v, sem_s)
    pltpu.sync_copy(p0_h.at[pl.ds(base, BPW)], p0_v)
    pltpu.sync_copy(i0_h.at[pl.ds(base, BPW)], i0_v)
    g1.wait()
    g2.wait()
    g3.wait()
    g4.wait()
    for jc in range(BPW // LANES):
        sl = pl.ds(jc * LANES, LANES)
        p0c = p0_v[sl]
        c1 = p1g_v[sl] > p0c
        best = jnp.where(c1, p1g_v[sl], p0c)
        bidx = jnp.where(c1, i1g_v[sl], i0_v[sl])
        c2 = p2g_v[sl] > best
        zsel_v[sl] = jnp.where(c2, i2g_v[sl], bidx)
    cp_a = pltpu.async_copy(cb0h_h.at[zsel_v], rows0_v, sem_a)
    pltpu.sync_copy(zsel_v, zout_h.at[pl.ds(base, BPW)])
    cp_a.wait()
    cp_b.wait()

    @plsc.parallel_loop(0, BPW, 1, unroll=4)
    def _add_row(r):
        for c in range(CH // LANES):
            s2 = pl.ds(c * LANES, LANES)
            rows0_v[r, s2] = (rows0_v[r, s2] + rows1_v[r, s2]) * 0.5

    pltpu.sync_copy(rows0_v, q_h.at[pl.ds(base, BPW)])


def _sc_combine(p0, i0, p1, i1, p2, i2, z2, m1, m2, cb0h, cb1h):
    mesh = plsc.VectorSubcoreMesh(core_axis_name="c", subcore_axis_name="s",
                                  num_cores=NC, num_subcores=NS)
    fn = pl.kernel(
        _sc_body,
        out_type=(jax.ShapeDtypeStruct((R0,), jnp.int32),
                  jax.ShapeDtypeStruct((R0, CH), jnp.float32)),
        mesh=mesh,
        scratch_types=[
            pltpu.VMEM((BPW,), jnp.float32),
            pltpu.VMEM((BPW,), jnp.int32),
            pltpu.VMEM((BPW,), jnp.int32),
            pltpu.VMEM((BPW,), jnp.int32),
            pltpu.VMEM((BPW,), jnp.int32),
            pltpu.VMEM((BPW,), jnp.float32),
            pltpu.VMEM((BPW,), jnp.int32),
            pltpu.VMEM((BPW,), jnp.float32),
            pltpu.VMEM((BPW,), jnp.int32),
            pltpu.VMEM((BPW,), jnp.int32),
            pltpu.VMEM((BPW, CH), jnp.float32),
            pltpu.VMEM((BPW, CH), jnp.float32),
            pltpu.SemaphoreType.DMA,
            pltpu.SemaphoreType.DMA,
            pltpu.SemaphoreType.DMA,
        ],
    )
    return fn(p0, i0, p1, i1, p2, i2, z2, m1, m2, cb0h, cb1h)


def kernel(input, cb0, cb1, cb2, cb3):
    b, c, h, w = input.shape
    r1 = jax.image.resize(input, (b, c, h // 2, w // 2), method='bilinear')
    r2 = jax.image.resize(input, (b, c, h // 4, w // 4), method='bilinear')
    x0 = input.reshape(B, CH, N0)
    x1 = r1.reshape(B, CH, N1)
    x2s = r2.reshape(B, CH, N2)

    norm0 = jnp.sum(jnp.square(jnp.transpose(input, (0, 2, 3, 1))),
                    axis=-1).reshape(B, 1, N0)
    norm1 = jnp.sum(jnp.square(jnp.transpose(r1, (0, 2, 3, 1))),
                    axis=-1).reshape(B, 1, N1)
    norm2 = jnp.sum(jnp.square(jnp.transpose(r2, (0, 2, 3, 1))),
                    axis=-1).reshape(B, 1, N2)
    c20 = jnp.sum(cb0 * cb0, axis=-1).reshape(K, 1)
    c21 = jnp.sum(cb1 * cb1, axis=-1).reshape(K, 1)

    p0, i0, p1, i1, p2, i2, zb = _tc_match(
        x0, x1, x2s, cb0, cb1, c20, c21, norm0, norm1, norm2)

    m1, m2 = _upsample_maps()
    zsel, q = _sc_combine(p0, i0, p1, i1, p2.reshape(R2), i2.reshape(R2),
                          zb, m1, m2, cb0, cb1)

    zidx0 = jnp.stack([zsel.reshape(b, h, w), zb.reshape(b, h, w)], axis=1)
    quant0 = jnp.transpose(q.reshape(b, h, w, CH), (0, 3, 1, 2))
    return input, zidx0, quant0


# speedup vs baseline: 4.4022x; 1.1900x over previous
import functools

import numpy as np
import jax
import jax.numpy as jnp
from jax import lax
from jax.experimental import pallas as pl
from jax.experimental.pallas import tpu as pltpu
from jax.experimental.pallas import tpu_sc as plsc

K = 1024
CH = 256
B = 4
H = W = 32
N0 = H * W
N1 = N0 // 4
N2 = N0 // 16
R0 = B * N0

NC, NS, LANES = 2, 16, 16
NW = NC * NS
BPW = R0 // NW

HIGHEST = lax.Precision.HIGHEST


def _match_body(x0_ref, x1_ref, x2s_ref, cb0_ref, cb1_ref, c20_ref, c21_ref,
                n0_ref, n1_ref, n2_ref, zsel_ref, zb_ref):
    def scale(xs, x2row, cb_ref, c2_ref, need_p):
        n = xs.shape[1]
        prod = lax.dot_general(cb_ref[...], xs, (((1,), (0,)), ((), ())),
                               preferred_element_type=jnp.float32)
        dist = x2row - 2.0 * prod + c2_ref[...]
        dmin = jnp.min(dist, axis=0, keepdims=True)
        ii = lax.broadcasted_iota(jnp.int32, (K, n), 0)
        idxrow = jnp.min(jnp.where(dist == dmin, ii, K), axis=0, keepdims=True)
        if not need_p:
            return None, idxrow
        denom = jnp.sum(jnp.exp(dmin - dist), axis=0, keepdims=True)
        return 1.0 / denom, idxrow

    p0, i0 = scale(x0_ref[0], n0_ref[0], cb0_ref, c20_ref, True)
    p1, i1 = scale(x1_ref[0], n1_ref[0], cb0_ref, c20_ref, True)
    p2, i2 = scale(x2s_ref[0], n2_ref[0], cb0_ref, c20_ref, True)
    _, zb = scale(x0_ref[0], n0_ref[0], cb1_ref, c21_ref, False)

    def up_onehot(n_coarse, n_fine, wsrc, wdst):
        q = lax.broadcasted_iota(jnp.int32, (1, n_fine), 1)
        m = (q // wdst // 2) * wsrc + (q % wdst) // 2
        r = lax.broadcasted_iota(jnp.int32, (n_coarse, n_fine), 0)
        return (r == m).astype(jnp.float32)

    u2 = up_onehot(N2, N1, W // 4, W // 2)
    s2u = lax.dot_general(jnp.concatenate([p2, i2.astype(jnp.float32)], axis=0),
                          u2, (((1,), (0,)), ((), ())),
                          precision=HIGHEST, preferred_element_type=jnp.float32)
    c12 = s2u[0:1] > p1
    pw = jnp.where(c12, s2u[0:1], p1)
    iwf = jnp.where(c12, s2u[1:2], i1.astype(jnp.float32))
    u1 = up_onehot(N1, N0, W // 2, W)
    s1u = lax.dot_general(jnp.concatenate([pw, iwf], axis=0), u1,
                          (((1,), (0,)), ((), ())),
                          precision=HIGHEST, preferred_element_type=jnp.float32)
    c01 = s1u[0:1] > p0
    iwu = (s1u[1:2] + 0.5).astype(jnp.int32)
    zsel_ref[...] = jnp.reshape(jnp.where(c01, iwu, i0), (N0,))
    zb_ref[...] = jnp.reshape(zb, (N0,))


def _tc_match(x0, x1, x2s, cb0, cb1, c20, c21, norm0, norm1, norm2):
    full = lambda shape: pl.BlockSpec(shape, lambda b: (0,) * len(shape))
    per_b = lambda shape: pl.BlockSpec((1,) + shape, lambda b: (b, 0, 0))
    lin = lambda n: pl.BlockSpec((n,), lambda b: (b,))
    return pl.pallas_call(
        _match_body,
        grid=(B,),
        in_specs=[
            per_b((CH, N0)), per_b((CH, N1)), per_b((CH, N2)),
            full((K, CH)), full((K, CH)), full((K, 1)), full((K, 1)),
            per_b((1, N0)), per_b((1, N1)), per_b((1, N2)),
        ],
        out_specs=[lin(N0), lin(N0)],
        out_shape=[
            jax.ShapeDtypeStruct((R0,), jnp.int32),
            jax.ShapeDtypeStruct((R0,), jnp.int32),
        ],
    )(x0, x1, x2s, cb0, cb1, c20, c21, norm0, norm1, norm2)


def _sc_body(zs_h, zb_h, cb0_h, cb1_h, q_h,
             zs_v, zb_v, rows0_v, rows1_v, sem_a, sem_b):
    wid = lax.axis_index("s") * NC + lax.axis_index("c")
    base = wid * BPW
    pltpu.sync_copy(zs_h.at[pl.ds(base, BPW)], zs_v)
    cp_a = pltpu.async_copy(cb0_h.at[zs_v], rows0_v, sem_a)
    pltpu.sync_copy(zb_h.at[pl.ds(base, BPW)], zb_v)
    cp_b = pltpu.async_copy(cb1_h.at[zb_v], rows1_v, sem_b)
    cp_a.wait()
    cp_b.wait()

    @plsc.parallel_loop(0, BPW, 1, unroll=4)
    def _add_row(r):
        for c in range(CH // LANES):
            s2 = pl.ds(c * LANES, LANES)
            rows0_v[r, s2] = (rows0_v[r, s2] + rows1_v[r, s2]) * 0.5

    pltpu.sync_copy(rows0_v, q_h.at[pl.ds(base, BPW)])


def _sc_combine(zsel, zb, cb0, cb1):
    mesh = plsc.VectorSubcoreMesh(core_axis_name="c", subcore_axis_name="s",
                                  num_cores=NC, num_subcores=NS)
    fn = pl.kernel(
        _sc_body,
        out_type=jax.ShapeDtypeStruct((R0, CH), jnp.float32),
        mesh=mesh,
        scratch_types=[
            pltpu.VMEM((BPW,), jnp.int32),
            pltpu.VMEM((BPW,), jnp.int32),
            pltpu.VMEM((BPW, CH), jnp.float32),
            pltpu.VMEM((BPW, CH), jnp.float32),
            pltpu.SemaphoreType.DMA,
            pltpu.SemaphoreType.DMA,
        ],
    )
    return fn(zsel, zb, cb0, cb1)


def kernel(input, cb0, cb1, cb2, cb3):
    b, c, h, w = input.shape
    r1 = jax.image.resize(input, (b, c, h // 2, w // 2), method='bilinear')
    r2 = jax.image.resize(input, (b, c, h // 4, w // 4), method='bilinear')
    x0 = input.reshape(B, CH, N0)
    x1 = r1.reshape(B, CH, N1)
    x2s = r2.reshape(B, CH, N2)

    norm0 = jnp.sum(jnp.square(jnp.transpose(input, (0, 2, 3, 1))),
                    axis=-1).reshape(B, 1, N0)
    norm1 = jnp.sum(jnp.square(jnp.transpose(r1, (0, 2, 3, 1))),
                    axis=-1).reshape(B, 1, N1)
    norm2 = jnp.sum(jnp.square(jnp.transpose(r2, (0, 2, 3, 1))),
                    axis=-1).reshape(B, 1, N2)
    c20 = jnp.sum(cb0 * cb0, axis=-1).reshape(K, 1)
    c21 = jnp.sum(cb1 * cb1, axis=-1).reshape(K, 1)

    zsel, zb = _tc_match(x0, x1, x2s, cb0, cb1, c20, c21, norm0, norm1, norm2)
    q = _sc_combine(zsel, zb, cb0, cb1)

    zidx0 = jnp.stack([zsel.reshape(b, h, w), zb.reshape(b, h, w)], axis=1)
    quant0 = jnp.transpose(q.reshape(b, h, w, CH), (0, 3, 1, 2))
    return input, zidx0, quant0


# speedup vs baseline: 4.4943x; 1.0209x over previous
import functools

import numpy as np
import jax
import jax.numpy as jnp
from jax import lax
from jax.experimental import pallas as pl
from jax.experimental.pallas import tpu as pltpu
from jax.experimental.pallas import tpu_sc as plsc

K = 1024
CH = 256
B = 4
H = W = 32
N0 = H * W
N1 = N0 // 4
N2 = N0 // 16
R0 = B * N0

NC, NS, LANES = 2, 16, 16
NW = NC * NS
BPW = R0 // NW

HIGHEST = lax.Precision.HIGHEST


def _fold_rows(t):
    sz = t.shape[0]
    while sz > 1:
        sz //= 2
        t = t[:sz] + t[sz:]
    return t


def _match_body(x0_ref, x1_ref, x2s_ref, cb0_ref, cb1_ref, c20_ref, c21_ref,
                zsel_ref, zb_ref):
    c20 = c20_ref[...]
    c21 = c21_ref[...]

    def scale(xs, cb_ref, c2, need_p):
        n = xs.shape[1]
        x2row = _fold_rows(xs * xs)
        prod = lax.dot_general(cb_ref[...], xs, (((1,), (0,)), ((), ())),
                               preferred_element_type=jnp.float32)
        dist = x2row - 2.0 * prod + c2
        dmin = jnp.min(dist, axis=0, keepdims=True)
        ii = lax.broadcasted_iota(jnp.int32, (K, n), 0)
        idxrow = jnp.min(jnp.where(dist == dmin, ii, K), axis=0, keepdims=True)
        if not need_p:
            return None, idxrow
        denom = jnp.sum(jnp.exp(dmin - dist), axis=0, keepdims=True)
        return 1.0 / denom, idxrow

    p0, i0 = scale(x0_ref[0], cb0_ref, c20, True)
    p1, i1 = scale(x1_ref[0], cb0_ref, c20, True)
    p2, i2 = scale(x2s_ref[0], cb0_ref, c20, True)
    _, zb = scale(x0_ref[0], cb1_ref, c21, False)

    def up_onehot(n_coarse, n_fine, wsrc, wdst):
        q = lax.broadcasted_iota(jnp.int32, (1, n_fine), 1)
        m = (q // wdst // 2) * wsrc + (q % wdst) // 2
        r = lax.broadcasted_iota(jnp.int32, (n_coarse, n_fine), 0)
        return (r == m).astype(jnp.float32)

    u2 = up_onehot(N2, N1, W // 4, W // 2)
    s2u = lax.dot_general(jnp.concatenate([p2, i2.astype(jnp.float32)], axis=0),
                          u2, (((1,), (0,)), ((), ())),
                          precision=HIGHEST, preferred_element_type=jnp.float32)
    c12 = s2u[0:1] > p1
    pw = jnp.where(c12, s2u[0:1], p1)
    iwf = jnp.where(c12, s2u[1:2], i1.astype(jnp.float32))
    u1 = up_onehot(N1, N0, W // 2, W)
    s1u = lax.dot_general(jnp.concatenate([pw, iwf], axis=0), u1,
                          (((1,), (0,)), ((), ())),
                          precision=HIGHEST, preferred_element_type=jnp.float32)
    c01 = s1u[0:1] > p0
    iwu = (s1u[1:2] + 0.5).astype(jnp.int32)
    zsel_ref[...] = jnp.reshape(jnp.where(c01, iwu, i0), (N0,))
    zb_ref[...] = jnp.reshape(zb, (N0,))


def _tc_match(x0, x1, x2s, cb0, cb1, c20, c21):
    full = lambda shape: pl.BlockSpec(shape, lambda b: (0,) * len(shape))
    per_b = lambda shape: pl.BlockSpec((1,) + shape, lambda b: (b, 0, 0))
    lin = lambda n: pl.BlockSpec((n,), lambda b: (b,))
    return pl.pallas_call(
        _match_body,
        grid=(B,),
        in_specs=[
            per_b((CH, N0)), per_b((CH, N1)), per_b((CH, N2)),
            full((K, CH)), full((K, CH)), full((K, 1)), full((K, 1)),
        ],
        out_specs=[lin(N0), lin(N0)],
        out_shape=[
            jax.ShapeDtypeStruct((R0,), jnp.int32),
            jax.ShapeDtypeStruct((R0,), jnp.int32),
        ],
    )(x0, x1, x2s, cb0, cb1, c20, c21)


def _sc_body(zs_h, zb_h, cb0_h, cb1_h, q_h,
             zs_v, zb_v, rows0_v, rows1_v, sem_a, sem_b):
    wid = lax.axis_index("s") * NC + lax.axis_index("c")
    base = wid * BPW
    pltpu.sync_copy(zs_h.at[pl.ds(base, BPW)], zs_v)
    cp_a = pltpu.async_copy(cb0_h.at[zs_v], rows0_v, sem_a)
    pltpu.sync_copy(zb_h.at[pl.ds(base, BPW)], zb_v)
    cp_b = pltpu.async_copy(cb1_h.at[zb_v], rows1_v, sem_b)
    cp_a.wait()
    cp_b.wait()

    @plsc.parallel_loop(0, BPW, 1, unroll=4)
    def _add_row(r):
        for c in range(CH // LANES):
            s2 = pl.ds(c * LANES, LANES)
            rows0_v[r, s2] = (rows0_v[r, s2] + rows1_v[r, s2]) * 0.5

    pltpu.sync_copy(rows0_v, q_h.at[pl.ds(base, BPW)])


def _sc_combine(zsel, zb, cb0, cb1):
    mesh = plsc.VectorSubcoreMesh(core_axis_name="c", subcore_axis_name="s",
                                  num_cores=NC, num_subcores=NS)
    fn = pl.kernel(
        _sc_body,
        out_type=jax.ShapeDtypeStruct((R0, CH), jnp.float32),
        mesh=mesh,
        scratch_types=[
            pltpu.VMEM((BPW,), jnp.int32),
            pltpu.VMEM((BPW,), jnp.int32),
            pltpu.VMEM((BPW, CH), jnp.float32),
            pltpu.VMEM((BPW, CH), jnp.float32),
            pltpu.SemaphoreType.DMA,
            pltpu.SemaphoreType.DMA,
        ],
    )
    return fn(zsel, zb, cb0, cb1)


def kernel(input, cb0, cb1, cb2, cb3):
    b, c, h, w = input.shape
    r1 = jax.image.resize(input, (b, c, h // 2, w // 2), method='bilinear')
    r2 = jax.image.resize(input, (b, c, h // 4, w // 4), method='bilinear')
    x0 = input.reshape(B, CH, N0)
    x1 = r1.reshape(B, CH, N1)
    x2s = r2.reshape(B, CH, N2)

    c20 = jnp.sum(cb0 * cb0, axis=-1).reshape(K, 1)
    c21 = jnp.sum(cb1 * cb1, axis=-1).reshape(K, 1)
    zsel, zb = _tc_match(x0, x1, x2s, cb0, cb1, c20, c21)
    q = _sc_combine(zsel, zb, cb0, cb1)

    zidx0 = jnp.stack([zsel.reshape(b, h, w), zb.reshape(b, h, w)], axis=1)
    quant0 = jnp.transpose(q.reshape(b, h, w, CH), (0, 3, 1, 2))
    return input, zidx0, quant0
